# Initial kernel scaffold; baseline (speedup 1.0000x reference)
import jax
import jax.numpy as jnp
from jax.experimental import pallas as pl


def kernel(x, edge_index, W1, b1, W2, b2):
    raise NotImplementedError("write your pallas kernel here")


# speedup vs baseline: 12.7105x; 12.7105x over previous
import functools

import jax
import jax.numpy as jnp
from jax import lax
from jax.experimental import pallas as pl
from jax.experimental.pallas import tpu as pltpu
from jax.experimental.pallas import tpu_sc as plsc

N = 10000
D = 128
NC = 2
NS = 16
NW = NC * NS
N_PAD = 10240
STRIPE = N_PAD // NS
E = 320000 + N
K = 128
G = -(-E // (NW * K))
E_PAD = NW * K * G
DEGC = 16
RB = 2000
NB = N // RB

_mesh = plsc.VectorSubcoreMesh(core_axis_name="c", subcore_axis_name="s")


@functools.partial(
    pl.kernel,
    out_type=jax.ShapeDtypeStruct((NC, N_PAD, DEGC), jnp.float32),
    mesh=_mesh,
    scratch_types=[
        pltpu.VMEM((K,), jnp.int32),
        pltpu.VMEM((K, DEGC), jnp.float32),
        pltpu.VMEM_SHARED((N_PAD, DEGC), jnp.float32),
    ],
)
def _sc_deg(dst_hbm, ones_hbm, zdeg_hbm, out_hbm, dst_v, ones_v, acc_sh):
    c = lax.axis_index("c")
    s = lax.axis_index("s")
    pltpu.sync_copy(zdeg_hbm, acc_sh.at[pl.ds(s * STRIPE, STRIPE)])
    pltpu.sync_copy(ones_hbm, ones_v)
    plsc.subcore_barrier()
    base0 = (c * NS + s) * (G * K)

    def body(g, carry):
        pltpu.sync_copy(dst_hbm.at[pl.ds(base0 + g * K, K)], dst_v)
        pltpu.sync_copy(ones_v, acc_sh.at[dst_v], add=True)
        return carry

    lax.fori_loop(0, G, body, 0)
    plsc.subcore_barrier()
    pltpu.sync_copy(acc_sh.at[pl.ds(s * STRIPE, STRIPE)],
                    out_hbm.at[c, pl.ds(s * STRIPE, STRIPE)])


@functools.partial(
    pl.kernel,
    out_type=jax.ShapeDtypeStruct((NC, N_PAD, D), jnp.float32),
    mesh=_mesh,
    scratch_types=[
        pltpu.VMEM((K,), jnp.int32),
        pltpu.VMEM((K,), jnp.int32),
        pltpu.VMEM((K, D), jnp.float32),
        pltpu.VMEM_SHARED((N_PAD, D), jnp.float32),
        pltpu.SemaphoreType.DMA,
    ],
)
def _sc_agg(h_hbm, src_hbm, dst_hbm, zrow_hbm, out_hbm,
            src_v, dst_v, rows_v, acc_sh, sem):
    c = lax.axis_index("c")
    s = lax.axis_index("s")
    pltpu.sync_copy(zrow_hbm, acc_sh.at[pl.ds(s * STRIPE, STRIPE)])
    plsc.subcore_barrier()
    base0 = (c * NS + s) * (G * K)

    def body(g, carry):
        base = base0 + g * K
        pltpu.sync_copy(src_hbm.at[pl.ds(base, K)], src_v)
        pltpu.sync_copy(dst_hbm.at[pl.ds(base, K)], dst_v)
        pltpu.async_copy(h_hbm.at[src_v], rows_v, sem).wait()
        pltpu.sync_copy(rows_v, acc_sh.at[dst_v], add=True)
        return carry

    lax.fori_loop(0, G, body, 0)
    plsc.subcore_barrier()
    pltpu.sync_copy(acc_sh.at[pl.ds(s * STRIPE, STRIPE)],
                    out_hbm.at[c, pl.ds(s * STRIPE, STRIPE)])


def _dinv(degp_ref):
    return lax.rsqrt(degp_ref[0, :, :1] + degp_ref[1, :, :1])


def _tc_in_body(x_ref, w_ref, degp_ref, o_ref):
    o_ref[...] = _dinv(degp_ref) * jnp.dot(
        x_ref[...], w_ref[...], preferred_element_type=jnp.float32)


_tc_in = pl.pallas_call(
    _tc_in_body,
    grid=(NB,),
    in_specs=[
        pl.BlockSpec((RB, D), lambda i: (i, 0)),
        pl.BlockSpec((D, D), lambda i: (0, 0)),
        pl.BlockSpec((NC, RB, DEGC), lambda i: (0, i, 0)),
    ],
    out_specs=pl.BlockSpec((RB, D), lambda i: (i, 0)),
    out_shape=jax.ShapeDtypeStruct((N, D), jnp.float32),
)


def _tc_mid_body(p_ref, degp_ref, b1_ref, w2_ref, o_ref):
    dinv = _dinv(degp_ref)
    h2 = jnp.maximum(dinv * (p_ref[0] + p_ref[1]) + b1_ref[...], 0.0)
    o_ref[...] = dinv * jnp.dot(h2, w2_ref[...],
                                preferred_element_type=jnp.float32)


_tc_mid = pl.pallas_call(
    _tc_mid_body,
    grid=(NB,),
    in_specs=[
        pl.BlockSpec((NC, RB, D), lambda i: (0, i, 0)),
        pl.BlockSpec((NC, RB, DEGC), lambda i: (0, i, 0)),
        pl.BlockSpec((1, D), lambda i: (0, 0)),
        pl.BlockSpec((D, D), lambda i: (0, 0)),
    ],
    out_specs=pl.BlockSpec((RB, D), lambda i: (i, 0)),
    out_shape=jax.ShapeDtypeStruct((N, D), jnp.float32),
)


def _tc_out_body(q_ref, degp_ref, b2_ref, o_ref):
    o_ref[...] = _dinv(degp_ref) * (q_ref[0] + q_ref[1]) + b2_ref[...]


_tc_out = pl.pallas_call(
    _tc_out_body,
    grid=(NB,),
    in_specs=[
        pl.BlockSpec((NC, RB, D), lambda i: (0, i, 0)),
        pl.BlockSpec((NC, RB, DEGC), lambda i: (0, i, 0)),
        pl.BlockSpec((1, D), lambda i: (0, 0)),
    ],
    out_specs=pl.BlockSpec((RB, D), lambda i: (i, 0)),
    out_shape=jax.ShapeDtypeStruct((N, D), jnp.float32),
)


def kernel(x, edge_index, W1, b1, W2, b2):
    ei = edge_index.astype(jnp.int32)
    loop = jnp.arange(N, dtype=jnp.int32)
    pad = E_PAD - E
    src = jnp.concatenate([ei[0], loop, jnp.zeros((pad,), jnp.int32)])
    dst = jnp.concatenate([ei[1], loop, jnp.full((pad,), N, jnp.int32)])
    ones_blk = jnp.ones((K, DEGC), jnp.float32)
    zdeg = jnp.zeros((STRIPE, DEGC), jnp.float32)
    zrow = jnp.zeros((STRIPE, D), jnp.float32)

    degp = _sc_deg(dst, ones_blk, zdeg)
    h1 = _tc_in(x, W1, degp)
    p = _sc_agg(h1, src, dst, zrow)
    h3 = _tc_mid(p, degp, b1.reshape(1, D), W2)
    q = _sc_agg(h3, src, dst, zrow)
    return _tc_out(q, degp, b2.reshape(1, D))
